# baseline (speedup 1.0000x reference)
import functools

import jax
import jax.numpy as jnp
from jax import lax
from jax.experimental import pallas as pl
from jax.experimental.pallas import tpu as pltpu
from jax.experimental.pallas import tpu_sc as plsc

_B = 16384
_L = 200
_D = 32

_info = plsc.get_sparse_core_info()
_NC, _NS = _info.num_cores, _info.num_subcores
_NW = _NC * _NS
_PER_W = _B // _NW
_NIDX = 8
_NROW = 4
_TPITCH = _PER_W + 1
_RUNROLL = 8

_mesh = plsc.VectorSubcoreMesh(core_axis_name="c", subcore_axis_name="s")


@functools.partial(
    pl.kernel,
    mesh=_mesh,
    out_type=jax.ShapeDtypeStruct((_L, _D, _B), jnp.float32),
    scratch_types=[
        pltpu.VMEM((_NIDX, _PER_W), jnp.int32),
        pltpu.VMEM((_NROW, _PER_W, _D), jnp.float32),
        pltpu.VMEM((2, _D, _TPITCH), jnp.float32),
        pltpu.SemaphoreType.DMA,
        pltpu.SemaphoreType.DMA,
        pltpu.SemaphoreType.DMA,
    ],
    compiler_params=pltpu.CompilerParams(
        use_tc_tiling_on_sc=False, needs_layout_passes=False,
        disable_bounds_checks=True),
)
def _gather_all(actt_hbm, table_hbm, out_hbm, idx_v, rows_v, t_v, isem, gsem, osem):
    wid = lax.axis_index("s") * _NC + lax.axis_index("c")
    base = wid * _PER_W
    iota = lax.iota(jnp.int32, 16)

    def idx_cp(l):
        return pltpu.make_async_copy(
            actt_hbm.at[l, pl.ds(base, _PER_W)], idx_v.at[l % _NIDX], isem)

    def gat_cp(l, rb):
        return pltpu.make_async_copy(
            table_hbm.at[idx_v.at[l % _NIDX]], rows_v.at[rb], gsem)

    def out_cp(l, tb):
        return pltpu.make_async_copy(
            t_v.at[tb, :, pl.ds(0, _PER_W)],
            out_hbm.at[l, :, pl.ds(base, _PER_W)], osem)

    def transpose(rb, tb):
        rref = rows_v.at[rb]
        tref = t_v.at[tb]
        dlo = iota
        dhi = iota + 16

        def rows8(i, carry):
            for u in range(_RUNROLL):
                r = i * _RUNROLL + u
                rv = jnp.full((16,), r, jnp.int32)
                plsc.store_scatter(tref, [dlo, rv], rref[r, pl.ds(0, 16)])
                plsc.store_scatter(tref, [dhi, rv], rref[r, pl.ds(16, 16)])
            return carry

        lax.fori_loop(0, _PER_W // _RUNROLL, rows8, 0)

    for i in range(_NIDX):
        idx_cp(i).start()
    for i in range(_NROW - 1):
        idx_cp(i).wait()
        gat_cp(i, i).start()

    def body(g, carry):
        for j in range(_NROW):
            l = _NROW * g + j
            rb = j
            tb = j % 2

            gat_cp(l, rb).wait()

            @pl.when(l >= 2)
            def _():
                out_cp(l - 2, tb).wait()

            transpose(rb, tb)
            out_cp(l, tb).start()

            @pl.when(l + _NROW - 1 < _L)
            def _():
                idx_cp(l + _NROW - 1).wait()
                gat_cp(l + _NROW - 1, (rb + _NROW - 1) % _NROW).start()

            @pl.when(l + _NIDX < _L)
            def _():
                idx_cp(l + _NIDX).start()

        return carry

    lax.fori_loop(0, _L // _NROW, body, 0)

    out_cp(_L - 2, 0).wait()
    out_cp(_L - 1, 1).wait()


def kernel(actions, table):
    actt = jnp.transpose(actions.astype(jnp.int32))
    out = _gather_all(actt, table)
    return jnp.transpose(out, (2, 0, 1))
